# baseline (speedup 1.0000x reference)
import functools

import jax
import jax.numpy as jnp
import numpy as np
from jax import lax
from jax.experimental import pallas as pl
from jax.experimental.pallas import tpu as pltpu
from jax.experimental.pallas import tpu_sc as plsc

N_NODES = 10000
N_EDGES = 320000
D_FEAT = 128
OUT_DIM = 64

NC = 2
NS = 16
NW = NC * NS
L = 16

DAUG = 144
DPAD = DAUG - D_FEAT
GCOL = 160
GW = GCOL // 2
N_GPAD = 10016
N_PAD = 10240
CHUNK = 80
NCHUNK = 128
NBATCH = NCHUNK // 2
EPW = CHUNK * NCHUNK
E_PAD = NW * EPW
RPT = N_PAD // NS


def _sc_segment_kernel(xtab_hbm, idx_hbm, wrows_hbm,
                       msg_out, deg_out, wsum_out,
                       idx_v, rbf0_v, rbf1_v, frows_v, wbuf_v,
                       acc_s, accw_s, semg0, semg1):
    c = lax.axis_index("c")
    s = lax.axis_index("s")
    wid = c * NS + s

    zeros16 = jnp.zeros((L,), jnp.float32)

    def _zero_bufs(i, _):
        for k in range(D_FEAT // L):
            frows_v[i, pl.ds(k * L, L)] = zeros16
        frows_v[i, pl.ds(DAUG - L, L)] = zeros16
        wbuf_v[i, pl.ds(0, L)] = zeros16
        wbuf_v[i + CHUNK, pl.ds(0, L)] = zeros16
        return 0

    lax.fori_loop(0, CHUNK, _zero_bufs, 0)

    r0 = s * RPT
    for k in range(RPT // CHUNK):
        pltpu.sync_copy(frows_v, acc_s.at[pl.ds(r0 + k * CHUNK, CHUNK)])
    for k in range(RPT // (2 * CHUNK)):
        pltpu.sync_copy(wbuf_v,
                        accw_s.at[pl.ds(r0 + k * 2 * CHUNK, 2 * CHUNK)])
    plsc.subcore_barrier()

    def _expand(rbf_v):
        def _row(r, _):
            for g in range(D_FEAT // 32):
                v = rbf_v[r, pl.ds(32 * g, 32)]
                a, b = plsc.unpack(v, format=plsc.PackFormat.INTERLEAVED)
                frows_v[r, pl.ds(2 * g * L, L)] = a
                frows_v[r, pl.ds((2 * g + 1) * L, L)] = b
            v = rbf_v[r, pl.ds(D_FEAT, 32)]
            a, _b = plsc.unpack(v, format=plsc.PackFormat.INTERLEAVED)
            frows_v[r, pl.ds(D_FEAT, L)] = a
            return 0

        lax.fori_loop(0, CHUNK, _row, 0)

    def _body(i, _):
        pltpu.sync_copy(idx_hbm.at[wid, i], idx_v)
        g0 = pltpu.async_copy(xtab_hbm.at[idx_v.at[0, 0]], rbf0_v, semg0)
        g1 = pltpu.async_copy(xtab_hbm.at[idx_v.at[1, 0]], rbf1_v, semg1)
        pltpu.sync_copy(wrows_hbm.at[wid, i], wbuf_v)
        g0.wait()
        _expand(rbf0_v)
        pltpu.sync_copy(frows_v, acc_s.at[idx_v.at[0, 1]], add=True)
        pltpu.sync_copy(wbuf_v.at[pl.ds(0, CHUNK)],
                        accw_s.at[idx_v.at[0, 0]], add=True)
        g1.wait()
        _expand(rbf1_v)
        pltpu.sync_copy(frows_v, acc_s.at[idx_v.at[1, 1]], add=True)
        pltpu.sync_copy(wbuf_v.at[pl.ds(CHUNK, CHUNK)],
                        accw_s.at[idx_v.at[1, 0]], add=True)
        return 0

    lax.fori_loop(0, NBATCH, _body, 0)
    plsc.subcore_barrier()

    row0 = c * N_PAD + r0
    pltpu.sync_copy(acc_s.at[pl.ds(r0, RPT), pl.ds(0, D_FEAT)],
                    msg_out.at[pl.ds(row0, RPT)])
    pltpu.sync_copy(acc_s.at[pl.ds(r0, RPT), pl.ds(D_FEAT, DPAD)],
                    deg_out.at[pl.ds(row0, RPT)])
    pltpu.sync_copy(accw_s.at[pl.ds(r0, RPT)],
                    wsum_out.at[pl.ds(row0, RPT)])


def _sc_aggregate(x_tab, idx_r, w_rows):
    mesh = plsc.VectorSubcoreMesh(core_axis_name="c", subcore_axis_name="s")
    f32 = jnp.float32
    return pl.kernel(
        _sc_segment_kernel,
        out_type=(
            jax.ShapeDtypeStruct((NC * N_PAD, D_FEAT), f32),
            jax.ShapeDtypeStruct((NC * N_PAD, DPAD), f32),
            jax.ShapeDtypeStruct((NC * N_PAD, L), f32),
        ),
        mesh=mesh,
        compiler_params=pltpu.CompilerParams(use_tc_tiling_on_sc=False,
                                             needs_layout_passes=False),
        scratch_types=[
            pltpu.VMEM((2, 2, CHUNK), jnp.int32),
            pltpu.VMEM((CHUNK, GCOL), jnp.bfloat16),
            pltpu.VMEM((CHUNK, GCOL), jnp.bfloat16),
            pltpu.VMEM((CHUNK, DAUG), f32),
            pltpu.VMEM((2 * CHUNK, L), f32),
            pltpu.VMEM_SHARED((N_PAD, DAUG), f32),
            pltpu.VMEM_SHARED((N_PAD, L), f32),
            pltpu.SemaphoreType.DMA,
            pltpu.SemaphoreType.DMA,
        ],
    )(x_tab, idx_r, w_rows)


TC_BLK = 1000
TC_GRID = N_NODES // TC_BLK


def _tc_finish_kernel(x_ref, msg_ref, deg_ref, wsum_ref,
                      w1t_ref, w2t_ref, bias_ref,
                      h_ref, reg_ref, vec_acc, s1_acc):
    i = pl.program_id(0)

    @pl.when(i == 0)
    def _init():
        vec_acc[...] = jnp.zeros_like(vec_acc)
        s1_acc[0] = 0.0

    x = x_ref[...]
    msg = msg_ref[0] + msg_ref[1]
    deg = (deg_ref[0] + deg_ref[1])[:, 0]
    y = msg / jnp.maximum(deg, 1.0)[:, None]
    h = (jnp.dot(x, w1t_ref[...], preferred_element_type=jnp.float32)
         + jnp.dot(y, w2t_ref[...], preferred_element_type=jnp.float32)
         + bias_ref[...])
    h_ref[...] = h

    a = (wsum_ref[0] + wsum_ref[1])[:, 0] * (1.0 / N_NODES)
    vec_acc[0, :] += jnp.sum(y, axis=0)
    vec_acc[1, :] += jnp.sum(a[:, None] * x, axis=0)
    s1_acc[0] += jnp.sum((a * a) * jnp.sum(x * x, axis=1))

    @pl.when(i == TC_GRID - 1)
    def _fin():
        m = vec_acc[0, :] * (1.0 / N_NODES)
        v = vec_acc[1, :]
        reg = (s1_acc[0] - 2.0 * jnp.sum(v * m)
               + N_NODES * jnp.sum(m * m)) / (N_NODES * OUT_DIM)
        reg_ref[...] = jnp.reshape(reg, (1, 1))


def _tc_finish(x, msg2, deg2, wsum2, w1t, w2t, bias2d):
    f32 = jnp.float32
    return pl.pallas_call(
        _tc_finish_kernel,
        grid=(TC_GRID,),
        in_specs=[
            pl.BlockSpec((TC_BLK, D_FEAT), lambda i: (i, 0)),
            pl.BlockSpec((NC, TC_BLK, D_FEAT), lambda i: (0, i, 0)),
            pl.BlockSpec((NC, TC_BLK, DPAD), lambda i: (0, i, 0)),
            pl.BlockSpec((NC, TC_BLK, L), lambda i: (0, i, 0)),
            pl.BlockSpec((D_FEAT, OUT_DIM), lambda i: (0, 0)),
            pl.BlockSpec((D_FEAT, OUT_DIM), lambda i: (0, 0)),
            pl.BlockSpec((1, OUT_DIM), lambda i: (0, 0)),
        ],
        out_specs=[
            pl.BlockSpec((TC_BLK, OUT_DIM), lambda i: (i, 0)),
            pl.BlockSpec((1, 1), lambda i: (0, 0)),
        ],
        out_shape=[
            jax.ShapeDtypeStruct((N_NODES, OUT_DIM), f32),
            jax.ShapeDtypeStruct((1, 1), f32),
        ],
        scratch_shapes=[
            pltpu.VMEM((2, D_FEAT), f32),
            pltpu.SMEM((1,), f32),
        ],
    )(x, msg2, deg2, wsum2, w1t, w2t, bias2d)


def kernel(edge_index, x, w, W_weight, W_bias):
    src = edge_index[0]
    dst = edge_index[1]

    pad = E_PAD - N_EDGES
    src_p = jnp.concatenate([src, jnp.full((pad,), N_NODES, jnp.int32)])
    dst_p = jnp.concatenate([dst, jnp.full((pad,), N_NODES, jnp.int32)])
    w_p = jnp.concatenate([w, jnp.zeros((pad,), jnp.float32)])
    idx_r = jnp.stack([src_p.reshape(NW, NBATCH, 2, CHUNK),
                       dst_p.reshape(NW, NBATCH, 2, CHUNK)], axis=3)
    w_rows = jnp.pad(w_p[:, None], ((0, 0), (0, L - 1))).reshape(
        NW, NBATCH, 2 * CHUNK, L)

    pm = np.empty((GCOL,), np.int32)
    for g in range(GCOL // 32):
        for i in range(L):
            pm[32 * g + 2 * i] = 32 * g + i
            pm[32 * g + 2 * i + 1] = 32 * g + L + i
    aug16 = jnp.zeros((N_GPAD, GCOL), jnp.bfloat16)
    aug16 = aug16.at[:N_NODES, :D_FEAT].set(x.astype(jnp.bfloat16))
    aug16 = aug16.at[:N_NODES, D_FEAT].set(1.0)
    x_tab = jnp.take(aug16, jnp.asarray(pm), axis=1)

    msg2, deg2, wsum2 = _sc_aggregate(x_tab, idx_r, w_rows)
    msg2 = msg2.reshape(NC, N_PAD, D_FEAT)
    deg2 = deg2.reshape(NC, N_PAD, DPAD)
    wsum2 = wsum2.reshape(NC, N_PAD, L)

    w1t = W_weight[:, :D_FEAT].T
    w2t = W_weight[:, D_FEAT:].T
    bias2d = W_bias[None, :]

    h, reg = _tc_finish(x, msg2, deg2, wsum2, w1t, w2t, bias2d)
    return (h, reg[0, 0])


# speedup vs baseline: 1.2643x; 1.2643x over previous
import functools

import jax
import jax.numpy as jnp
import numpy as np
from jax import lax
from jax.experimental import pallas as pl
from jax.experimental.pallas import tpu as pltpu
from jax.experimental.pallas import tpu_sc as plsc

N_NODES = 10000
N_EDGES = 320000
D_FEAT = 128
OUT_DIM = 64

NC = 2
NS = 16
NW = NC * NS
L = 16

DAUG = 144
DPAD = DAUG - D_FEAT
GCOL = 160
GW = GCOL // 2
N_GPAD = 10016
N_PAD = 10240
CHUNK = 80
NCHUNK = 128
NBATCH = NCHUNK // 2
EPW = CHUNK * NCHUNK
E_PAD = NW * EPW
RPT = N_PAD // NS


def _sc_segment_kernel(xtab_hbm, idx_hbm, wrows_hbm,
                       msg_out, deg_out, wsum_out,
                       idx_v, rbf0_v, rbf1_v, frows_v, wbuf_v,
                       acc_s, accw_s, semg0, semg1):
    c = lax.axis_index("c")
    s = lax.axis_index("s")
    wid = c * NS + s

    zeros16 = jnp.zeros((L,), jnp.float32)

    def _zero_bufs(i, _):
        for k in range(D_FEAT // L):
            frows_v[i, pl.ds(k * L, L)] = zeros16
        frows_v[i, pl.ds(DAUG - L, L)] = zeros16
        wbuf_v[i, pl.ds(0, L)] = zeros16
        wbuf_v[i + CHUNK, pl.ds(0, L)] = zeros16
        return 0

    lax.fori_loop(0, CHUNK, _zero_bufs, 0)

    r0 = s * RPT
    for k in range(RPT // CHUNK):
        pltpu.sync_copy(frows_v, acc_s.at[pl.ds(r0 + k * CHUNK, CHUNK)])
    for k in range(RPT // (2 * CHUNK)):
        pltpu.sync_copy(wbuf_v,
                        accw_s.at[pl.ds(r0 + k * 2 * CHUNK, 2 * CHUNK)])
    plsc.subcore_barrier()

    def _expand(rbf_v):
        @functools.partial(plsc.parallel_loop, 0, CHUNK, unroll=4)
        def _row(r):
            for g in range(D_FEAT // 32):
                v = rbf_v[r, pl.ds(32 * g, 32)]
                a, b = plsc.unpack(v, format=plsc.PackFormat.INTERLEAVED)
                frows_v[r, pl.ds(2 * g * L, L)] = a
                frows_v[r, pl.ds((2 * g + 1) * L, L)] = b
            v = rbf_v[r, pl.ds(D_FEAT, 32)]
            a, _b = plsc.unpack(v, format=plsc.PackFormat.INTERLEAVED)
            frows_v[r, pl.ds(D_FEAT, L)] = a

    def _body(i, _):
        pltpu.sync_copy(idx_hbm.at[wid, i], idx_v)
        g0 = pltpu.async_copy(xtab_hbm.at[idx_v.at[0, 0]], rbf0_v, semg0)
        g1 = pltpu.async_copy(xtab_hbm.at[idx_v.at[1, 0]], rbf1_v, semg1)
        pltpu.sync_copy(wrows_hbm.at[wid, i], wbuf_v)
        g0.wait()
        _expand(rbf0_v)
        pltpu.sync_copy(frows_v, acc_s.at[idx_v.at[0, 1]], add=True)
        pltpu.sync_copy(wbuf_v.at[pl.ds(0, CHUNK)],
                        accw_s.at[idx_v.at[0, 0]], add=True)
        g1.wait()
        _expand(rbf1_v)
        pltpu.sync_copy(frows_v, acc_s.at[idx_v.at[1, 1]], add=True)
        pltpu.sync_copy(wbuf_v.at[pl.ds(CHUNK, CHUNK)],
                        accw_s.at[idx_v.at[1, 0]], add=True)
        return 0

    lax.fori_loop(0, NBATCH, _body, 0)
    plsc.subcore_barrier()

    row0 = c * N_PAD + r0
    pltpu.sync_copy(acc_s.at[pl.ds(r0, RPT), pl.ds(0, D_FEAT)],
                    msg_out.at[pl.ds(row0, RPT)])
    pltpu.sync_copy(acc_s.at[pl.ds(r0, RPT), pl.ds(D_FEAT, DPAD)],
                    deg_out.at[pl.ds(row0, RPT)])
    pltpu.sync_copy(accw_s.at[pl.ds(r0, RPT)],
                    wsum_out.at[pl.ds(row0, RPT)])


def _sc_aggregate(x_tab, idx_r, w_rows):
    mesh = plsc.VectorSubcoreMesh(core_axis_name="c", subcore_axis_name="s")
    f32 = jnp.float32
    return pl.kernel(
        _sc_segment_kernel,
        out_type=(
            jax.ShapeDtypeStruct((NC * N_PAD, D_FEAT), f32),
            jax.ShapeDtypeStruct((NC * N_PAD, DPAD), f32),
            jax.ShapeDtypeStruct((NC * N_PAD, L), f32),
        ),
        mesh=mesh,
        compiler_params=pltpu.CompilerParams(use_tc_tiling_on_sc=False,
                                             needs_layout_passes=False),
        scratch_types=[
            pltpu.VMEM((2, 2, CHUNK), jnp.int32),
            pltpu.VMEM((CHUNK, GCOL), jnp.bfloat16),
            pltpu.VMEM((CHUNK, GCOL), jnp.bfloat16),
            pltpu.VMEM((CHUNK, DAUG), f32),
            pltpu.VMEM((2 * CHUNK, L), f32),
            pltpu.VMEM_SHARED((N_PAD, DAUG), f32),
            pltpu.VMEM_SHARED((N_PAD, L), f32),
            pltpu.SemaphoreType.DMA,
            pltpu.SemaphoreType.DMA,
        ],
    )(x_tab, idx_r, w_rows)


TC_BLK = 1000
TC_GRID = N_NODES // TC_BLK


def _tc_finish_kernel(x_ref, msg_ref, deg_ref, wsum_ref,
                      w1t_ref, w2t_ref, bias_ref,
                      h_ref, reg_ref, vec_acc, s1_acc):
    i = pl.program_id(0)

    @pl.when(i == 0)
    def _init():
        vec_acc[...] = jnp.zeros_like(vec_acc)
        s1_acc[0] = 0.0

    x = x_ref[...]
    msg = msg_ref[0] + msg_ref[1]
    deg = (deg_ref[0] + deg_ref[1])[:, 0]
    y = msg / jnp.maximum(deg, 1.0)[:, None]
    h = (jnp.dot(x, w1t_ref[...], preferred_element_type=jnp.float32)
         + jnp.dot(y, w2t_ref[...], preferred_element_type=jnp.float32)
         + bias_ref[...])
    h_ref[...] = h

    a = (wsum_ref[0] + wsum_ref[1])[:, 0] * (1.0 / N_NODES)
    vec_acc[0, :] += jnp.sum(y, axis=0)
    vec_acc[1, :] += jnp.sum(a[:, None] * x, axis=0)
    s1_acc[0] += jnp.sum((a * a) * jnp.sum(x * x, axis=1))

    @pl.when(i == TC_GRID - 1)
    def _fin():
        m = vec_acc[0, :] * (1.0 / N_NODES)
        v = vec_acc[1, :]
        reg = (s1_acc[0] - 2.0 * jnp.sum(v * m)
               + N_NODES * jnp.sum(m * m)) / (N_NODES * OUT_DIM)
        reg_ref[...] = jnp.reshape(reg, (1, 1))


def _tc_finish(x, msg2, deg2, wsum2, w1t, w2t, bias2d):
    f32 = jnp.float32
    return pl.pallas_call(
        _tc_finish_kernel,
        grid=(TC_GRID,),
        in_specs=[
            pl.BlockSpec((TC_BLK, D_FEAT), lambda i: (i, 0)),
            pl.BlockSpec((NC, TC_BLK, D_FEAT), lambda i: (0, i, 0)),
            pl.BlockSpec((NC, TC_BLK, DPAD), lambda i: (0, i, 0)),
            pl.BlockSpec((NC, TC_BLK, L), lambda i: (0, i, 0)),
            pl.BlockSpec((D_FEAT, OUT_DIM), lambda i: (0, 0)),
            pl.BlockSpec((D_FEAT, OUT_DIM), lambda i: (0, 0)),
            pl.BlockSpec((1, OUT_DIM), lambda i: (0, 0)),
        ],
        out_specs=[
            pl.BlockSpec((TC_BLK, OUT_DIM), lambda i: (i, 0)),
            pl.BlockSpec((1, 1), lambda i: (0, 0)),
        ],
        out_shape=[
            jax.ShapeDtypeStruct((N_NODES, OUT_DIM), f32),
            jax.ShapeDtypeStruct((1, 1), f32),
        ],
        scratch_shapes=[
            pltpu.VMEM((2, D_FEAT), f32),
            pltpu.SMEM((1,), f32),
        ],
    )(x, msg2, deg2, wsum2, w1t, w2t, bias2d)


def kernel(edge_index, x, w, W_weight, W_bias):
    src = edge_index[0]
    dst = edge_index[1]

    pad = E_PAD - N_EDGES
    src_p = jnp.concatenate([src, jnp.full((pad,), N_NODES, jnp.int32)])
    dst_p = jnp.concatenate([dst, jnp.full((pad,), N_NODES, jnp.int32)])
    w_p = jnp.concatenate([w, jnp.zeros((pad,), jnp.float32)])
    idx_r = jnp.stack([src_p.reshape(NW, NBATCH, 2, CHUNK),
                       dst_p.reshape(NW, NBATCH, 2, CHUNK)], axis=3)
    w_rows = jnp.pad(w_p[:, None], ((0, 0), (0, L - 1))).reshape(
        NW, NBATCH, 2 * CHUNK, L)

    pm = np.empty((GCOL,), np.int32)
    for g in range(GCOL // 32):
        for i in range(L):
            pm[32 * g + 2 * i] = 32 * g + i
            pm[32 * g + 2 * i + 1] = 32 * g + L + i
    aug16 = jnp.zeros((N_GPAD, GCOL), jnp.bfloat16)
    aug16 = aug16.at[:N_NODES, :D_FEAT].set(x.astype(jnp.bfloat16))
    aug16 = aug16.at[:N_NODES, D_FEAT].set(1.0)
    x_tab = jnp.take(aug16, jnp.asarray(pm), axis=1)

    msg2, deg2, wsum2 = _sc_aggregate(x_tab, idx_r, w_rows)
    msg2 = msg2.reshape(NC, N_PAD, D_FEAT)
    deg2 = deg2.reshape(NC, N_PAD, DPAD)
    wsum2 = wsum2.reshape(NC, N_PAD, L)

    w1t = W_weight[:, :D_FEAT].T
    w2t = W_weight[:, D_FEAT:].T
    bias2d = W_bias[None, :]

    h, reg = _tc_finish(x, msg2, deg2, wsum2, w1t, w2t, bias2d)
    return (h, reg[0, 0])
